# baseline (speedup 1.0000x reference)
import functools

import jax
import jax.numpy as jnp
from jax import lax
from jax.experimental import pallas as pl
from jax.experimental.pallas import tpu as pltpu
from jax.experimental.pallas import tpu_sc as plsc

NUM_CLASSES = 1000
HIDDEN = 128
BATCH = 16384

_info = plsc.get_sparse_core_info()
_NC, _NS = _info.num_cores, _info.num_subcores
_NW = _NC * _NS
_BPW = BATCH // _NW
_CHUNK = 64
_NCHUNK = _BPW // _CHUNK
_NHBM = 4

_mesh = plsc.VectorSubcoreMesh(core_axis_name="c", subcore_axis_name="s")


@functools.partial(
    pl.kernel,
    mesh=_mesh,
    out_type=jax.ShapeDtypeStruct((BATCH, HIDDEN), jnp.float32),
    scratch_types=[
        pltpu.VMEM((_BPW // 128, 128), jnp.int32),
        pltpu.VMEM((_BPW, HIDDEN), jnp.float32),
        pltpu.VMEM_SHARED((NUM_CLASSES + 1, HIDDEN), jnp.float32),
        [pltpu.SemaphoreType.DMA for _ in range(_NCHUNK)],
        pltpu.SemaphoreType.DMA,
        pltpu.SemaphoreType.DMA,
    ],
)
def _sc_embed(table_hbm, labels_hbm, out_hbm, idx_v, rows_v, tab_sh,
              gsems, wsem, ssem):
    sid = lax.axis_index("s")
    wid = lax.axis_index("c") * _NS + sid
    start = pl.multiple_of(sid * 64, 8)
    stage64 = pltpu.make_async_copy(
        table_hbm.at[pl.ds(start, 64)], tab_sh.at[pl.ds(start, 64)], ssem)
    stage41 = pltpu.make_async_copy(
        table_hbm.at[pl.ds(960, NUM_CLASSES + 1 - 960)],
        tab_sh.at[pl.ds(960, NUM_CLASSES + 1 - 960)], ssem)

    @pl.when(sid < 15)
    def _():
        stage64.start()

    @pl.when(sid == 15)
    def _():
        stage41.start()

    pltpu.sync_copy(labels_hbm.at[wid], idx_v)

    gathers = []
    for j in range(_NHBM):
        gathers.append(
            pltpu.async_copy(
                table_hbm.at[idx_v.at[j // 2, pl.ds((j % 2) * _CHUNK, _CHUNK)]],
                rows_v.at[pl.ds(j * _CHUNK, _CHUNK)],
                gsems[j],
            )
        )

    @pl.when(sid < 15)
    def _():
        stage64.wait()

    @pl.when(sid == 15)
    def _():
        stage41.wait()

    plsc.subcore_barrier()
    for j in range(_NHBM, _NCHUNK):
        gathers.append(
            pltpu.async_copy(
                tab_sh.at[idx_v.at[j // 2, pl.ds((j % 2) * _CHUNK, _CHUNK)]],
                rows_v.at[pl.ds(j * _CHUNK, _CHUNK)],
                gsems[j],
            )
        )
    writes = []
    for j in range(_NCHUNK):
        gathers[j].wait()
        writes.append(
            pltpu.async_copy(
                rows_v.at[pl.ds(j * _CHUNK, _CHUNK)],
                out_hbm.at[pl.ds(wid * _BPW + j * _CHUNK, _CHUNK)],
                wsem,
            )
        )
    for w in writes:
        w.wait()


def kernel(labels, train, dtype, table):
    labels3d = labels.astype(jnp.int32).reshape(_NW, _BPW // 128, 128)
    out = _sc_embed(table, labels3d)
    return out.astype(dtype.dtype)


# speedup vs baseline: 1.1270x; 1.1270x over previous
import functools

import jax
import jax.numpy as jnp
from jax import lax
from jax.experimental import pallas as pl
from jax.experimental.pallas import tpu as pltpu
from jax.experimental.pallas import tpu_sc as plsc

NUM_CLASSES = 1000
HIDDEN = 128
BATCH = 16384

_info = plsc.get_sparse_core_info()
_NC, _NS = _info.num_cores, _info.num_subcores
_NW = _NC * _NS
_BPW = BATCH // _NW
_CHUNK = 64
_NCHUNK = _BPW // _CHUNK
_NHBM = 1

_mesh = plsc.VectorSubcoreMesh(core_axis_name="c", subcore_axis_name="s")


@functools.partial(
    pl.kernel,
    mesh=_mesh,
    out_type=jax.ShapeDtypeStruct((BATCH, HIDDEN), jnp.float32),
    scratch_types=[
        pltpu.VMEM((_BPW // 128, 128), jnp.int32),
        pltpu.VMEM((_BPW, HIDDEN), jnp.float32),
        pltpu.VMEM_SHARED((NUM_CLASSES + 1, HIDDEN), jnp.float32),
        [pltpu.SemaphoreType.DMA for _ in range(_NCHUNK)],
        pltpu.SemaphoreType.DMA,
        pltpu.SemaphoreType.DMA,
    ],
)
def _sc_embed(table_hbm, labels_hbm, out_hbm, idx_v, rows_v, tab_sh,
              gsems, wsem, ssem):
    sid = lax.axis_index("s")
    wid = lax.axis_index("c") * _NS + sid
    start = pl.multiple_of(sid * 64, 8)
    stage64 = pltpu.make_async_copy(
        table_hbm.at[pl.ds(start, 64)], tab_sh.at[pl.ds(start, 64)], ssem)
    stage41 = pltpu.make_async_copy(
        table_hbm.at[pl.ds(960, NUM_CLASSES + 1 - 960)],
        tab_sh.at[pl.ds(960, NUM_CLASSES + 1 - 960)], ssem)

    @pl.when(sid < 15)
    def _():
        stage64.start()

    @pl.when(sid == 15)
    def _():
        stage41.start()

    pltpu.sync_copy(labels_hbm.at[wid], idx_v)

    gathers = []
    for j in range(_NHBM):
        gathers.append(
            pltpu.async_copy(
                table_hbm.at[idx_v.at[j // 2, pl.ds((j % 2) * _CHUNK, _CHUNK)]],
                rows_v.at[pl.ds(j * _CHUNK, _CHUNK)],
                gsems[j],
            )
        )

    @pl.when(sid < 15)
    def _():
        stage64.wait()

    @pl.when(sid == 15)
    def _():
        stage41.wait()

    plsc.subcore_barrier()
    for j in range(_NHBM, _NCHUNK):
        gathers.append(
            pltpu.async_copy(
                tab_sh.at[idx_v.at[j // 2, pl.ds((j % 2) * _CHUNK, _CHUNK)]],
                rows_v.at[pl.ds(j * _CHUNK, _CHUNK)],
                gsems[j],
            )
        )
    writes = []
    for j in range(_NCHUNK):
        gathers[j].wait()
        writes.append(
            pltpu.async_copy(
                rows_v.at[pl.ds(j * _CHUNK, _CHUNK)],
                out_hbm.at[pl.ds(wid * _BPW + j * _CHUNK, _CHUNK)],
                wsem,
            )
        )
    for w in writes:
        w.wait()


def kernel(labels, train, dtype, table):
    labels3d = labels.astype(jnp.int32).reshape(_NW, _BPW // 128, 128)
    out = _sc_embed(table, labels3d)
    return out.astype(dtype.dtype)
